# Initial kernel scaffold; baseline (speedup 1.0000x reference)
import functools

import jax
import jax.numpy as jnp
from jax import lax
from jax.experimental import pallas as pl
from jax.experimental.pallas import tpu as pltpu
from jax.experimental.pallas import tpu_sc as plsc

_RANK = 8
_DIM = 64
_LANES = 16
_CHUNK = 128


def _lora_embed_body(steps, x_hbm, w_hbm, a_hbm, bs_hbm, out_hbm,
                     idx_v, arows_v, wrows_v, bs_v, sem_a, sem_w):
  num_cores = 2
  cid = lax.axis_index("c")
  sid = lax.axis_index("s")
  wid = sid * num_cores + cid

  pltpu.sync_copy(x_hbm.at[pl.ds(wid * steps, steps)], idx_v)
  pltpu.sync_copy(bs_hbm, bs_v)

  bsv = [[bs_v[r, pl.ds(j * _LANES, _LANES)] for j in range(4)]
         for r in range(_RANK)]

  def step(t, carry):
    idx_row = idx_v.at[t]
    cp_a = pltpu.async_copy(a_hbm.at[idx_row], arows_v, sem_a)
    cp_w = pltpu.async_copy(w_hbm.at[idx_row], wrows_v, sem_w)
    cp_a.wait()
    cp_w.wait()

    def row(i, c):
      asp = [plsc.load_gather(
          arows_v, [jnp.full((_LANES,), 0, jnp.int32) + i,
                    jnp.full((_LANES,), r, jnp.int32)])
             for r in range(_RANK)]
      for j in range(4):
        acc = wrows_v[i, pl.ds(j * _LANES, _LANES)]
        for r in range(_RANK):
          acc = acc + asp[r] * bsv[r][j]
        wrows_v[i, pl.ds(j * _LANES, _LANES)] = acc
      return c

    lax.fori_loop(0, _CHUNK, row, 0, unroll=2)
    pltpu.sync_copy(wrows_v,
                    out_hbm.at[pl.ds((wid * steps + t) * _CHUNK, _CHUNK)])
    return carry

  lax.fori_loop(0, steps, step, 0)


def kernel(x, weight, lora_A, lora_B):
  scaling = _RANK / (_RANK ** 0.5)
  n = x.shape[0] * x.shape[1]
  num_workers = 32
  rows_per_w = n // num_workers
  steps = rows_per_w // _CHUNK

  xf = x.reshape(n).astype(jnp.int32).reshape(num_workers * steps, _CHUNK)
  bs = (lora_B * scaling).astype(jnp.float32)

  mesh = plsc.VectorSubcoreMesh(core_axis_name="c", subcore_axis_name="s")
  run = pl.kernel(
      functools.partial(_lora_embed_body, steps),
      out_type=jax.ShapeDtypeStruct((n, _DIM), jnp.float32),
      mesh=mesh,
      scratch_types=[
          pltpu.VMEM((steps, _CHUNK), jnp.int32),
          pltpu.VMEM((_CHUNK, _RANK), jnp.float32),
          pltpu.VMEM((_CHUNK, _DIM), jnp.float32),
          pltpu.VMEM((_RANK, _DIM), jnp.float32),
          pltpu.SemaphoreType.DMA,
          pltpu.SemaphoreType.DMA,
      ],
  )
  out = run(xf, weight, lora_A, bs)
  return out.reshape(x.shape[0], x.shape[1], _DIM)


# speedup vs baseline: 2.6474x; 2.6474x over previous
import functools

import jax
import jax.numpy as jnp
from jax import lax
from jax.experimental import pallas as pl
from jax.experimental.pallas import tpu as pltpu
from jax.experimental.pallas import tpu_sc as plsc

_RANK = 8
_DIM = 64
_LANES = 16
_CHUNK = 128


def _lora_embed_body(steps, x_hbm, xe_hbm, w_hbm, aflat_hbm, bs_hbm, out_hbm,
                     idx_v, xe_v, aflat_v, wrows_v, bs_v, sem_a, sem_w):
  num_cores = 2
  cid = lax.axis_index("c")
  sid = lax.axis_index("s")
  wid = sid * num_cores + cid

  pltpu.sync_copy(x_hbm.at[pl.ds(wid * steps, steps)], idx_v)
  pltpu.sync_copy(bs_hbm, bs_v)

  bsv = [[bs_v[r, pl.ds(j * _LANES, _LANES)] for j in range(4)]
         for r in range(_RANK)]
  cidx = [jnp.full((_LANES,), l, jnp.int32) for l in range(_LANES)]

  def step(t, carry):
    chunk = wid * steps + t
    idx_row = idx_v.at[t]
    pltpu.sync_copy(xe_hbm.at[chunk], xe_v)
    cp_w = pltpu.async_copy(w_hbm.at[idx_row], wrows_v, sem_w)
    cps = [pltpu.async_copy(aflat_hbm.at[xe_v.at[k]], aflat_v.at[k], sem_a)
           for k in range(_RANK)]
    for cp in cps:
      cp.wait()
    cp_w.wait()

    def krow(k, c):
      for j in range(8):
        ap = aflat_v[k, pl.ds(16 * j, _LANES)]
        asp = [ap.at[cidx[l]].get(mode="promise_in_bounds")
               for l in range(_LANES)]
        for half in range(2):
          row = 16 * k + 2 * j + half
          for jj in range(4):
            acc = wrows_v[row, pl.ds(jj * _LANES, _LANES)]
            for r in range(_RANK):
              acc = acc + asp[8 * half + r] * bsv[r][jj]
            wrows_v[row, pl.ds(jj * _LANES, _LANES)] = acc
      return c

    lax.fori_loop(0, _CHUNK // 16, krow, 0)
    pltpu.sync_copy(wrows_v, out_hbm.at[pl.ds(chunk * _CHUNK, _CHUNK)])
    return carry

  lax.fori_loop(0, steps, step, 0)


def kernel(x, weight, lora_A, lora_B):
  scaling = _RANK / (_RANK ** 0.5)
  n = x.shape[0] * x.shape[1]
  num_workers = 32
  rows_per_w = n // num_workers
  steps = rows_per_w // _CHUNK
  nchunks = num_workers * steps

  xf = x.reshape(n).astype(jnp.int32).reshape(nchunks, _CHUNK)
  xe = (xf[:, :, None] * _RANK
        + jnp.arange(_RANK, dtype=jnp.int32)).reshape(nchunks, _RANK, _CHUNK)
  bs = (lora_B * scaling).astype(jnp.float32)
  a_flat = lora_A.reshape(lora_A.shape[0] * _RANK)

  mesh = plsc.VectorSubcoreMesh(core_axis_name="c", subcore_axis_name="s")
  run = pl.kernel(
      functools.partial(_lora_embed_body, steps),
      out_type=jax.ShapeDtypeStruct((n, _DIM), jnp.float32),
      mesh=mesh,
      compiler_params=pltpu.CompilerParams(use_tc_tiling_on_sc=False),
      scratch_types=[
          pltpu.VMEM((steps, _CHUNK), jnp.int32),
          pltpu.VMEM((_RANK, _CHUNK), jnp.int32),
          pltpu.VMEM((_RANK, _CHUNK), jnp.float32),
          pltpu.VMEM((_CHUNK, _DIM), jnp.float32),
          pltpu.VMEM((_RANK, _DIM), jnp.float32),
          pltpu.SemaphoreType.DMA,
          pltpu.SemaphoreType.DMA,
      ],
  )
  out = run(xf, xe, weight, a_flat, bs)
  return out.reshape(x.shape[0], x.shape[1], _DIM)
